# Initial kernel scaffold; baseline (speedup 1.0000x reference)
import jax
import jax.numpy as jnp
from jax.experimental import pallas as pl


def kernel(hidden_states, router_w, expert_weights, expert_mapping):
    raise NotImplementedError("write your pallas kernel here")


# speedup vs baseline: 1.0950x; 1.0950x over previous
import functools

import jax
import jax.numpy as jnp
from jax.experimental import pallas as pl
from jax.experimental.pallas import tpu as pltpu

H, E = 1024, 8
BLK = 256


def _moe_body(x_ref, rw_ref, w_ref, out_ref, i1_ref, i2_ref, w1_ref, w2_ref):
    e = pl.program_id(1)

    @pl.when(e == 0)
    def _router():
        x = x_ref[...]
        logits = jnp.dot(x, rw_ref[...].T, preferred_element_type=jnp.float32)
        m = jnp.max(logits, axis=-1, keepdims=True)
        p = jnp.exp(logits - m)
        probs = p / jnp.sum(p, axis=-1, keepdims=True)
        iota = jax.lax.broadcasted_iota(jnp.int32, probs.shape, 1)
        m1 = jnp.max(probs, axis=-1, keepdims=True)
        i1 = jnp.min(jnp.where(probs == m1, iota, E), axis=-1, keepdims=True)
        probs2 = jnp.where(iota == i1, -jnp.inf, probs)
        m2 = jnp.max(probs2, axis=-1, keepdims=True)
        i2 = jnp.min(jnp.where(probs2 == m2, iota, E), axis=-1, keepdims=True)
        denom = m1 + m2 + 1e-6
        w1 = m1 / denom
        w2 = m2 / denom
        i1_ref[...] = i1
        i2_ref[...] = i2
        w1_ref[...] = w1
        w2_ref[...] = w2
        out_ref[...] = jnp.broadcast_to(w1 * m1 + w2 * m2, out_ref.shape)

    c = (w1_ref[...] * (i1_ref[...] == e).astype(jnp.float32)
         + w2_ref[...] * (i2_ref[...] == e).astype(jnp.float32))
    out_ref[...] += c * jnp.dot(x_ref[...], w_ref[0],
                                preferred_element_type=jnp.float32)


@jax.jit
def _moe(x, router_w, expert_weights):
    n = x.shape[0]
    grid = (n // BLK, E)
    return pl.pallas_call(
        _moe_body,
        grid=grid,
        in_specs=[
            pl.BlockSpec((BLK, H), lambda i, e: (i, 0)),
            pl.BlockSpec((E, H), lambda i, e: (0, 0)),
            pl.BlockSpec((1, H, H), lambda i, e: (e, 0, 0)),
        ],
        out_specs=pl.BlockSpec((BLK, H), lambda i, e: (i, 0)),
        out_shape=jax.ShapeDtypeStruct((n, H), jnp.float32),
        scratch_shapes=[
            pltpu.VMEM((BLK, 1), jnp.int32),
            pltpu.VMEM((BLK, 1), jnp.int32),
            pltpu.VMEM((BLK, 1), jnp.float32),
            pltpu.VMEM((BLK, 1), jnp.float32),
        ],
    )(x, router_w, expert_weights)


def kernel(hidden_states, router_w, expert_weights, expert_mapping):
    b, s, h = hidden_states.shape
    x = hidden_states.reshape(-1, h)
    out = _moe(x, router_w, expert_weights)
    return out.reshape(b, s, h)


# speedup vs baseline: 1.2632x; 1.1535x over previous
import functools

import jax
import jax.numpy as jnp
from jax.experimental import pallas as pl
from jax.experimental.pallas import tpu as pltpu

H, E = 1024, 8
BLK = 256


def _moe_body(x_ref, rw_ref, w_ref, out_ref, i1_ref, i2_ref, w1_ref, w2_ref):
    e = pl.program_id(1)

    @pl.when(e == 0)
    def _router():
        x = x_ref[...]
        logits = jnp.dot(x, rw_ref[...].T, preferred_element_type=jnp.float32)
        m = jnp.max(logits, axis=-1, keepdims=True)
        p = jnp.exp(logits - m)
        probs = p / jnp.sum(p, axis=-1, keepdims=True)
        iota = jax.lax.broadcasted_iota(jnp.int32, probs.shape, 1)
        m1 = jnp.max(probs, axis=-1, keepdims=True)
        i1 = jnp.min(jnp.where(probs == m1, iota, E), axis=-1, keepdims=True)
        probs2 = jnp.where(iota == i1, -jnp.inf, probs)
        m2 = jnp.max(probs2, axis=-1, keepdims=True)
        i2 = jnp.min(jnp.where(probs2 == m2, iota, E), axis=-1, keepdims=True)
        denom = m1 + m2 + 1e-6
        w1 = m1 / denom
        w2 = m2 / denom
        i1_ref[...] = i1
        i2_ref[...] = i2
        w1_ref[...] = w1
        w2_ref[...] = w2
        out_ref[...] = jnp.broadcast_to(w1 * m1 + w2 * m2, out_ref.shape)

    c = (w1_ref[...] * (i1_ref[...] == e).astype(jnp.float32)
         + w2_ref[...] * (i2_ref[...] == e).astype(jnp.float32))
    out_ref[...] += c * jnp.dot(x_ref[...].astype(jnp.bfloat16), w_ref[0],
                                preferred_element_type=jnp.float32)


@jax.jit
def _moe(x, router_w, expert_weights):
    n = x.shape[0]
    grid = (n // BLK, E)
    return pl.pallas_call(
        _moe_body,
        grid=grid,
        in_specs=[
            pl.BlockSpec((BLK, H), lambda i, e: (i, 0)),
            pl.BlockSpec((E, H), lambda i, e: (0, 0)),
            pl.BlockSpec((1, H, H), lambda i, e: (e, 0, 0)),
        ],
        out_specs=pl.BlockSpec((BLK, H), lambda i, e: (i, 0)),
        out_shape=jax.ShapeDtypeStruct((n, H), jnp.float32),
        scratch_shapes=[
            pltpu.VMEM((BLK, 1), jnp.int32),
            pltpu.VMEM((BLK, 1), jnp.int32),
            pltpu.VMEM((BLK, 1), jnp.float32),
            pltpu.VMEM((BLK, 1), jnp.float32),
        ],
    )(x, router_w, expert_weights)


def kernel(hidden_states, router_w, expert_weights, expert_mapping):
    b, s, h = hidden_states.shape
    x = hidden_states.reshape(-1, h)
    out = _moe(x, router_w, expert_weights.astype(jnp.bfloat16))
    return out.reshape(b, s, h)
